# baseline (speedup 1.0000x reference)
import jax
import jax.numpy as jnp
from jax.experimental import pallas as pl
from jax.experimental.pallas import tpu as pltpu


def _tc_body(f_ref, c_ref, out_ref, cb_ref):
    i = pl.program_id(0)
    nsteps = pl.num_programs(0)
    d = f_ref.shape[1]

    @pl.when(i == 0)
    def _():
        c = c_ref[...]
        csq = jnp.sum(c * c, axis=1, keepdims=True)
        csq_hi = csq.astype(jnp.bfloat16)
        csq_lo = (csq - csq_hi.astype(jnp.float32)).astype(jnp.bfloat16)
        ones_k = jnp.ones((c.shape[0], d), jnp.bfloat16)
        cb_ref[...] = jnp.concatenate(
            [(c * -2.0).astype(jnp.bfloat16), csq_hi, csq_lo, ones_k],
            axis=1)
        out_ref[0, 0] = 0.0

    fb = f_ref[...].astype(jnp.bfloat16)
    ones_n = jnp.ones((fb.shape[0], 2), jnp.bfloat16)
    faug = jnp.concatenate([fb, ones_n, fb * fb], axis=1)

    kb = 256
    parts = []
    for j in range(cb_ref.shape[0] // kb):
        sqc = jax.lax.dot_general(
            cb_ref[pl.ds(j * kb, kb), :], faug, (((1,), (1,)), ((), ())),
            preferred_element_type=jnp.float32)
        parts.append(jnp.min(sqc, axis=0, keepdims=True))
    minsq = jnp.min(jnp.concatenate(parts, axis=0), axis=0, keepdims=True)
    dist = jnp.minimum(jnp.sqrt(jnp.maximum(minsq, 0.0)), 1000000.0)
    out_ref[0, 0] += jnp.sum(dist)

    @pl.when(i == nsteps - 1)
    def _():
        out_ref[0, 0] = out_ref[0, 0] * (1.0 / (nsteps * fb.shape[0]))


def kernel(features, centers):
    n, d = features.shape
    k = centers.shape[0]
    bn = 2048

    out = pl.pallas_call(
        _tc_body,
        grid=(n // bn,),
        in_specs=[
            pl.BlockSpec((bn, d), lambda i: (i, 0)),
            pl.BlockSpec((k, d), lambda i: (0, 0)),
        ],
        out_specs=pl.BlockSpec((1, 1), lambda i: (0, 0),
                               memory_space=pltpu.SMEM),
        out_shape=jax.ShapeDtypeStruct((1, 1), jnp.float32),
        scratch_shapes=[pltpu.VMEM((k, 2 * d + 2), jnp.bfloat16)],
    )(features, centers)
    return out[0, 0]


# speedup vs baseline: 1.6177x; 1.6177x over previous
import jax
import jax.numpy as jnp
from jax.experimental import pallas as pl
from jax.experimental.pallas import tpu as pltpu


def _tc_body(ft_ref, c_ref, out_ref, cb_ref):
    i = pl.program_id(0)
    nsteps = pl.num_programs(0)

    @pl.when(i == 0)
    def _():
        c = c_ref[...]
        csq = jnp.sum(c * c, axis=1, keepdims=True)
        csq_hi = csq.astype(jnp.bfloat16)
        csq_lo = (csq - csq_hi.astype(jnp.float32)).astype(jnp.bfloat16)
        cb_ref[...] = jnp.concatenate(
            [(c * -2.0).astype(jnp.bfloat16), csq_hi, csq_lo], axis=1)
        out_ref[0, 0] = 0.0

    ft = ft_ref[...]
    fsq = jnp.sum(ft * ft, axis=0, keepdims=True)
    fb = ft.astype(jnp.bfloat16)
    ones2 = jnp.ones((2, ft.shape[1]), jnp.bfloat16)
    faug = jnp.concatenate([fb, ones2], axis=0)
    sq = jax.lax.dot_general(
        cb_ref[...], faug, (((1,), (0,)), ((), ())),
        preferred_element_type=jnp.float32)
    minsq = jnp.min(sq, axis=0, keepdims=True)
    dist = jnp.minimum(jnp.sqrt(jnp.maximum(minsq + fsq, 0.0)), 1000000.0)
    out_ref[0, 0] += jnp.sum(dist)

    @pl.when(i == nsteps - 1)
    def _():
        out_ref[0, 0] = out_ref[0, 0] * (1.0 / (nsteps * ft.shape[1]))


def kernel(features, centers):
    n, d = features.shape
    k = centers.shape[0]
    bn = 4096
    ft = features.T

    out = pl.pallas_call(
        _tc_body,
        grid=(n // bn,),
        in_specs=[
            pl.BlockSpec((d, bn), lambda i: (0, i)),
            pl.BlockSpec((k, d), lambda i: (0, 0)),
        ],
        out_specs=pl.BlockSpec((1, 1), lambda i: (0, 0),
                               memory_space=pltpu.SMEM),
        out_shape=jax.ShapeDtypeStruct((1, 1), jnp.float32),
        scratch_shapes=[pltpu.VMEM((k, d + 2), jnp.bfloat16)],
    )(ft, centers)
    return out[0, 0]


# speedup vs baseline: 1.7241x; 1.0657x over previous
import jax
import jax.numpy as jnp
from jax.experimental import pallas as pl
from jax.experimental.pallas import tpu as pltpu


def _tc_body(ft_ref, c_ref, out_ref, cb_ref):
    i = pl.program_id(0)
    nsteps = pl.num_programs(0)

    @pl.when(i == 0)
    def _():
        c = c_ref[...]
        csq = jnp.sum(c * c, axis=1, keepdims=True)
        csq_hi = csq.astype(jnp.bfloat16)
        csq_lo = (csq - csq_hi.astype(jnp.float32)).astype(jnp.bfloat16)
        cb_ref[...] = jnp.concatenate(
            [(c * -2.0).astype(jnp.bfloat16), csq_hi, csq_lo], axis=1)
        out_ref[0, 0] = 0.0

    ft = ft_ref[...]
    fsq = jnp.sum(ft * ft, axis=0, keepdims=True)
    fb = ft.astype(jnp.bfloat16)
    ones2 = jnp.ones((2, ft.shape[1]), jnp.bfloat16)
    faug = jnp.concatenate([fb, ones2], axis=0)
    kb = 256
    parts = []
    for j in range(cb_ref.shape[0] // kb):
        sqc = jax.lax.dot_general(
            cb_ref[pl.ds(j * kb, kb), :], faug, (((1,), (0,)), ((), ())),
            preferred_element_type=jnp.float32)
        parts.append(jnp.min(sqc, axis=0, keepdims=True))
    minsq = jnp.min(jnp.concatenate(parts, axis=0), axis=0, keepdims=True)
    dist = jnp.minimum(jnp.sqrt(jnp.maximum(minsq + fsq, 0.0)), 1000000.0)
    out_ref[0, 0] += jnp.sum(dist)

    @pl.when(i == nsteps - 1)
    def _():
        out_ref[0, 0] = out_ref[0, 0] * (1.0 / (nsteps * ft.shape[1]))


def kernel(features, centers):
    n, d = features.shape
    k = centers.shape[0]
    bn = 16384
    ft = features.T

    out = pl.pallas_call(
        _tc_body,
        grid=(n // bn,),
        in_specs=[
            pl.BlockSpec((d, bn), lambda i: (0, i)),
            pl.BlockSpec((k, d), lambda i: (0, 0)),
        ],
        out_specs=pl.BlockSpec((1, 1), lambda i: (0, 0),
                               memory_space=pltpu.SMEM),
        out_shape=jax.ShapeDtypeStruct((1, 1), jnp.float32),
        scratch_shapes=[pltpu.VMEM((k, d + 2), jnp.bfloat16)],
    )(ft, centers)
    return out[0, 0]
